# baseline (speedup 1.0000x reference)
import functools

import jax
import jax.numpy as jnp
import numpy as np
from jax import lax
from jax.experimental import pallas as pl
from jax.experimental.pallas import tpu as pltpu
from jax.experimental.pallas import tpu_sc as plsc

_EXPERT_TYPES = (0, 1, 2, 0, 1, 2, 0, 1)
_E = 8
_TOP_K = 2
_TBLK = 256

_INV_SQRT2 = float(1.0 / np.sqrt(2.0))


def _gelu(t):
    return 0.5 * t * (1.0 + lax.erf(t * _INV_SQRT2))


def _softplus(t):
    return jnp.maximum(t, 0.0) + jnp.log(1.0 + jnp.exp(-jnp.abs(t)))


def _bdot(a, b):
    return jnp.dot(a.astype(jnp.bfloat16), b,
                   preferred_element_type=jnp.float32)


def _prep_body(tf_ref, w1tT_ref, rb1_ref, c_ref):
    c_ref[...] = _bdot(tf_ref[...], w1tT_ref[...]) + rb1_ref[...]


def _main_body(x_ref, w1xT_ref, c_ref, w2T_ref, nW1T_ref, nb1_ref, nW2T_ref,
               nb2_ref, rb2_ref, noise_ref, ct_ref, noisy_ref):
    xb = x_ref[...]
    xp = _bdot(xb, w1xT_ref[...])
    cols = []
    for e in range(_E):
        h = _gelu(xp + c_ref[e:e + 1, :])
        le = _bdot(h, w2T_ref[...]) + rb2_ref[...]
        cols.append(jnp.mean(le, axis=1, keepdims=True))
    logits = jnp.concatenate(cols, axis=1)

    nh = _gelu(_bdot(xb, nW1T_ref[...]) + nb1_ref[...])
    nstd = _softplus(_softplus(_bdot(nh, nW2T_ref[...]) + nb2_ref[...]))

    ie = lax.broadcasted_iota(jnp.int32, (1, _E), 1)
    wide = jnp.zeros((1, _E), jnp.bool_)
    for j, ty in enumerate(_EXPERT_TYPES):
        if ty == 1:
            wide = wide | (ie == j)
    wbias = jnp.where(wide, 0.3, 0.0)
    ct = ct_ref[0, 0]
    noisy_ref[...] = logits + ct * (noise_ref[...] * nstd) + wbias


def _route_sc_body(logits_hbm, rout_hbm, idx_hbm, in_v, out_v, idx_v):
    cid = lax.axis_index("c")
    sid = lax.axis_index("s")
    wid = sid * 2 + cid
    tpw = in_v.shape[0] // _E
    base = wid * tpw * _E
    pltpu.sync_copy(logits_hbm.at[pl.ds(base, tpw * _E)], in_v)
    zero16 = jnp.zeros((16,), jnp.float32)
    for j in range(tpw * _E // 16):
        out_v[pl.ds(j * 16, 16)] = zero16
    for g in range(tpw // 16):
        tok = lax.broadcasted_iota(jnp.int32, (16,), 0) + g * 16
        rowbase = tok * _E
        ls = [plsc.load_gather(in_v, [rowbase + e]) for e in range(_E)]
        m1 = ls[0]
        for e in range(1, _E):
            m1 = jnp.maximum(m1, ls[e])
        i1 = jnp.zeros((16,), jnp.int32)
        for e in range(_E - 1, -1, -1):
            i1 = jnp.where(ls[e] == m1, e, i1)
        neginf = jnp.full((16,), -jnp.inf, jnp.float32)
        ms = [jnp.where(i1 == e, neginf, ls[e]) for e in range(_E)]
        m2 = ms[0]
        for e in range(1, _E):
            m2 = jnp.maximum(m2, ms[e])
        i2 = jnp.zeros((16,), jnp.int32)
        for e in range(_E - 1, -1, -1):
            i2 = jnp.where(ms[e] == m2, e, i2)
        r = jnp.exp(m2 - m1)
        denom = 1.0 + r
        p1 = 1.0 / denom
        p2 = r / denom
        plsc.store_scatter(out_v, [rowbase + i1], p1)
        plsc.store_scatter(out_v, [rowbase + i2], p2)
        plsc.store_scatter(idx_v, [tok * _TOP_K], i1)
        plsc.store_scatter(idx_v, [tok * _TOP_K + 1], i2)
    pltpu.sync_copy(out_v, rout_hbm.at[pl.ds(base, tpw * _E)])
    pltpu.sync_copy(idx_v, idx_hbm.at[pl.ds(wid * tpw * _TOP_K, tpw * _TOP_K)])


def kernel(x, type_emb, nW1, nb1, nW2, nb2, rW1, rb1, rW2, rb2, temperature):
    B, S, D = x.shape
    N = B * S
    F = rW1.shape[0]
    H = nW1.shape[0]
    x2 = x.reshape(N, D).astype(jnp.float32)

    et = jnp.asarray(np.array(_EXPERT_TYPES, dtype=np.int32))
    tf = jnp.take(type_emb, et, axis=0)
    bf16 = jnp.bfloat16
    W1xT = rW1[:, :D].T.astype(bf16)
    W1tT = rW1[:, D:].T.astype(bf16)
    w2T = rW2.T.astype(bf16)
    nW1T = nW1.T.astype(bf16)
    nW2T = nW2.T.astype(bf16)

    c = pl.pallas_call(
        _prep_body,
        out_shape=jax.ShapeDtypeStruct((_E, F), jnp.float32),
    )(tf, W1tT, rb1.reshape(1, F))

    noise = jax.random.normal(jax.random.key(42), (B, S, _E),
                              dtype=jnp.float32).reshape(N, _E)
    ct = jnp.clip(temperature * (0.95 ** (S // 100)), 0.05, 3.0)
    ct = ct.reshape(1, 1).astype(jnp.float32)

    nblk = N // _TBLK
    noisy = pl.pallas_call(
        _main_body,
        grid=(nblk,),
        in_specs=[
            pl.BlockSpec((_TBLK, D), lambda i: (i, 0)),
            pl.BlockSpec((D, F), lambda i: (0, 0)),
            pl.BlockSpec((_E, F), lambda i: (0, 0)),
            pl.BlockSpec((F, _E), lambda i: (0, 0)),
            pl.BlockSpec((D, H), lambda i: (0, 0)),
            pl.BlockSpec((1, H), lambda i: (0, 0)),
            pl.BlockSpec((H, _E), lambda i: (0, 0)),
            pl.BlockSpec((1, _E), lambda i: (0, 0)),
            pl.BlockSpec((1, _E), lambda i: (0, 0)),
            pl.BlockSpec((_TBLK, _E), lambda i: (i, 0)),
            pl.BlockSpec(memory_space=pltpu.SMEM),
        ],
        out_specs=pl.BlockSpec((_TBLK, _E), lambda i: (i, 0)),
        out_shape=jax.ShapeDtypeStruct((N, _E), jnp.float32),
    )(x2, W1xT, c, w2T, nW1T, nb1.reshape(1, H), nW2T, nb2.reshape(1, _E),
      rb2.reshape(1, _E), noise, ct)

    nw = 32
    tpw = N // nw
    route = pl.kernel(
        _route_sc_body,
        mesh=plsc.VectorSubcoreMesh(core_axis_name="c", subcore_axis_name="s"),
        compiler_params=pltpu.CompilerParams(needs_layout_passes=False),
        out_type=[
            jax.ShapeDtypeStruct((N * _E,), jnp.float32),
            jax.ShapeDtypeStruct((N * _TOP_K,), jnp.int32),
        ],
        scratch_types=[
            pltpu.VMEM((tpw * _E,), jnp.float32),
            pltpu.VMEM((tpw * _E,), jnp.float32),
            pltpu.VMEM((tpw * _TOP_K,), jnp.int32),
        ],
    )
    rout_flat, idx_flat = route(noisy.reshape(N * _E))

    return (rout_flat.reshape(B, S, _E).astype(x.dtype),
            idx_flat.reshape(B, S, _TOP_K))
